# baseline (speedup 1.0000x reference)
import functools

import jax
import jax.numpy as jnp
import numpy as np
from jax.experimental import pallas as pl
from jax.experimental.pallas import tpu as pltpu

N_ROWS = 128
N_COLS = 100000
BLOCK_C = 2048
N_BLOCKS = (N_COLS + BLOCK_C - 1) // BLOCK_C


def _rotl(x, d):
    return jax.lax.shift_left(x, np.int32(d)) | jax.lax.shift_right_logical(
        x, np.int32(32 - d)
    )


def _threefry2x32_bits(flat_idx):
    ks0 = jnp.int32(0)
    ks1 = jnp.int32(42)
    ks2 = ks0 ^ ks1 ^ jnp.int32(0x1BD11BDA)
    rot_a = (13, 15, 26, 6)
    rot_b = (17, 29, 16, 24)

    def four_rounds(x0, x1, rots):
        for r in rots:
            x0 = x0 + x1
            x1 = _rotl(x1, r)
            x1 = x0 ^ x1
        return x0, x1

    x0 = jnp.zeros_like(flat_idx) + ks0
    x1 = flat_idx + ks1
    x0, x1 = four_rounds(x0, x1, rot_a)
    x0 = x0 + ks1
    x1 = x1 + ks2 + jnp.int32(1)
    x0, x1 = four_rounds(x0, x1, rot_b)
    x0 = x0 + ks2
    x1 = x1 + ks0 + jnp.int32(2)
    x0, x1 = four_rounds(x0, x1, rot_a)
    x0 = x0 + ks0
    x1 = x1 + ks1 + jnp.int32(3)
    x0, x1 = four_rounds(x0, x1, rot_b)
    x0 = x0 + ks1
    x1 = x1 + ks2 + jnp.int32(4)
    x0, x1 = four_rounds(x0, x1, rot_a)
    x0 = x0 + ks2
    x1 = x1 + ks0 + jnp.int32(5)
    return x0 ^ x1


def _gumbel_noise(bits):
    fb = jax.lax.shift_right_logical(bits, np.int32(9)) | jnp.int32(0x3F800000)
    u = jax.lax.bitcast_convert_type(fb, jnp.float32) - 1.0
    return -jnp.log(-jnp.log(u + 1e-08) + 1e-08)


def _argmax_kernel(logits_ref, idx_ref, vmax_ref, vidx_ref):
    j = pl.program_id(0)
    col0 = j * BLOCK_C
    rows = jax.lax.broadcasted_iota(jnp.int32, (N_ROWS, BLOCK_C), 0)
    cols = jax.lax.broadcasted_iota(jnp.int32, (N_ROWS, BLOCK_C), 1) + col0
    bits = _threefry2x32_bits(rows * N_COLS + cols)
    noise = _gumbel_noise(bits)
    z = jnp.where(cols < N_COLS, logits_ref[...] + noise, -jnp.inf)
    bmax = jnp.max(z, axis=1, keepdims=True)
    bidx = jnp.min(
        jnp.where(z == bmax, cols, jnp.int32(0x7FFFFFFF)), axis=1, keepdims=True
    )

    @pl.when(j == 0)
    def _():
        vmax_ref[...] = bmax
        vidx_ref[...] = bidx

    @pl.when(j > 0)
    def _():
        better = bmax > vmax_ref[...]
        vidx_ref[...] = jnp.where(better, bidx, vidx_ref[...])
        vmax_ref[...] = jnp.where(better, bmax, vmax_ref[...])

    @pl.when(j == pl.num_programs(0) - 1)
    def _():
        idx_ref[...] = vidx_ref[...]


def _onehot_kernel(idx_ref, out_ref):
    j = pl.program_id(0)
    cols = jax.lax.broadcasted_iota(jnp.int32, (N_ROWS, BLOCK_C), 1) + j * BLOCK_C
    out_ref[...] = (cols == idx_ref[...]).astype(jnp.float32)


@functools.partial(jax.jit, donate_argnums=())
def kernel(logits):
    idx = pl.pallas_call(
        _argmax_kernel,
        grid=(N_BLOCKS,),
        in_specs=[pl.BlockSpec((N_ROWS, BLOCK_C), lambda j: (0, j))],
        out_specs=pl.BlockSpec((N_ROWS, 1), lambda j: (0, 0)),
        out_shape=jax.ShapeDtypeStruct((N_ROWS, 1), jnp.int32),
        scratch_shapes=[
            pltpu.VMEM((N_ROWS, 1), jnp.float32),
            pltpu.VMEM((N_ROWS, 1), jnp.int32),
        ],
    )(logits)
    out = pl.pallas_call(
        _onehot_kernel,
        grid=(N_BLOCKS,),
        in_specs=[pl.BlockSpec((N_ROWS, 1), lambda j: (0, 0))],
        out_specs=pl.BlockSpec((N_ROWS, BLOCK_C), lambda j: (0, j)),
        out_shape=jax.ShapeDtypeStruct((N_ROWS, N_COLS), jnp.float32),
    )(idx)
    return out


# speedup vs baseline: 1.5189x; 1.5189x over previous
import jax
import jax.numpy as jnp
import numpy as np
from jax.experimental import pallas as pl
from jax.experimental.pallas import tpu as pltpu

N_ROWS = 128
N_COLS = 100000
BLK_V = 4096
N_BLOCKS = (N_COLS + BLK_V - 1) // BLK_V


def _rotl(x, d):
    return jax.lax.shift_left(x, np.int32(d)) | jax.lax.shift_right_logical(
        x, np.int32(32 - d)
    )


def _threefry2x32_bits(flat_idx):
    ks0 = jnp.int32(0)
    ks1 = jnp.int32(42)
    ks2 = ks0 ^ ks1 ^ jnp.int32(0x1BD11BDA)
    rot_a = (13, 15, 26, 6)
    rot_b = (17, 29, 16, 24)

    def four_rounds(x0, x1, rots):
        for r in rots:
            x0 = x0 + x1
            x1 = _rotl(x1, r)
            x1 = x0 ^ x1
        return x0, x1

    x1 = flat_idx + ks1
    x0 = x1
    x1 = _rotl(x1, rot_a[0]) ^ x0
    for r in rot_a[1:]:
        x0 = x0 + x1
        x1 = _rotl(x1, r)
        x1 = x0 ^ x1
    x0 = x0 + ks1
    x1 = x1 + ks2 + jnp.int32(1)
    x0, x1 = four_rounds(x0, x1, rot_b)
    x0 = x0 + ks2
    x1 = x1 + ks0 + jnp.int32(2)
    x0, x1 = four_rounds(x0, x1, rot_a)
    x0 = x0 + ks0
    x1 = x1 + ks1 + jnp.int32(3)
    x0, x1 = four_rounds(x0, x1, rot_b)
    x0 = x0 + ks1
    x1 = x1 + ks2 + jnp.int32(4)
    x0, x1 = four_rounds(x0, x1, rot_a)
    x0 = x0 + ks2
    x1 = x1 + ks0 + jnp.int32(5)
    return x0 ^ x1


def _gumbel_noise(bits):
    fb = jax.lax.shift_right_logical(bits, np.int32(9)) | jnp.int32(0x3F800000)
    u = jax.lax.bitcast_convert_type(fb, jnp.float32) - 1.0
    return -jnp.log(-jnp.log(u + 1e-08) + 1e-08)


def _argmax_kernel(lt_ref, idx_ref, zeros_ref, vmax_ref, vidx_ref):
    j = pl.program_id(0)
    v0 = j * BLK_V
    blk = lt_ref[...].T
    r_id = jax.lax.broadcasted_iota(jnp.int32, (N_ROWS, BLK_V), 0)
    v_id = jax.lax.broadcasted_iota(jnp.int32, (N_ROWS, BLK_V), 1) + v0
    bits = _threefry2x32_bits(r_id * N_COLS + v_id)
    noise = _gumbel_noise(bits)
    z = jnp.where(v_id < N_COLS, blk + noise, -jnp.inf)
    bmax = jnp.max(z, axis=1, keepdims=True)
    bidx = jnp.min(
        jnp.where(z == bmax, v_id, jnp.int32(0x7FFFFFFF)), axis=1, keepdims=True
    )
    zeros_ref[...] = jnp.zeros((BLK_V, N_ROWS), jnp.float32)

    @pl.when(j == 0)
    def _():
        vmax_ref[...] = bmax
        vidx_ref[...] = bidx

    @pl.when(j > 0)
    def _():
        better = bmax > vmax_ref[...]
        vidx_ref[...] = jnp.where(better, bidx, vidx_ref[...])
        vmax_ref[...] = jnp.where(better, bmax, vmax_ref[...])

    @pl.when(j == pl.num_programs(0) - 1)
    def _():
        idx_ref[...] = vidx_ref[...].T


def _patch_kernel(idx_sref, idx_ref, zeros_ref, out_ref, patch_ref, sem):
    del zeros_ref
    idxv = idx_ref[...]
    s_io = jax.lax.broadcasted_iota(jnp.int32, (8, N_ROWS), 0)

    def build(r, _):
        band_start = (idx_sref[r] // 8) * 8
        patch_ref[pl.ds(r * 8, 8), :] = (s_io + band_start == idxv).astype(
            jnp.float32
        )
        return 0

    jax.lax.fori_loop(0, N_ROWS, build, 0)

    def launch(r, _):
        band_start = (idx_sref[r] // 8) * 8
        pltpu.make_async_copy(
            patch_ref.at[pl.ds(r * 8, 8), :],
            out_ref.at[pl.ds(band_start, 8), :],
            sem,
        ).start()
        return 0

    jax.lax.fori_loop(0, N_ROWS, launch, 0)

    def wait(r, _):
        pltpu.make_async_copy(
            patch_ref.at[pl.ds(r * 8, 8), :],
            out_ref.at[pl.ds((idx_sref[r] // 8) * 8, 8), :],
            sem,
        ).wait()
        return 0

    jax.lax.fori_loop(0, N_ROWS, wait, 0)


def kernel(logits):
    lt = logits.T
    idx, zeros_t = pl.pallas_call(
        _argmax_kernel,
        grid=(N_BLOCKS,),
        in_specs=[pl.BlockSpec((BLK_V, N_ROWS), lambda j: (j, 0))],
        out_specs=[
            pl.BlockSpec((1, N_ROWS), lambda j: (0, 0)),
            pl.BlockSpec((BLK_V, N_ROWS), lambda j: (j, 0)),
        ],
        out_shape=[
            jax.ShapeDtypeStruct((1, N_ROWS), jnp.int32),
            jax.ShapeDtypeStruct((N_COLS, N_ROWS), jnp.float32),
        ],
        scratch_shapes=[
            pltpu.VMEM((N_ROWS, 1), jnp.float32),
            pltpu.VMEM((N_ROWS, 1), jnp.int32),
        ],
    )(lt)
    out_t = pl.pallas_call(
        _patch_kernel,
        in_specs=[
            pl.BlockSpec(memory_space=pltpu.SMEM),
            pl.BlockSpec(memory_space=pltpu.VMEM),
            pl.BlockSpec(memory_space=pl.ANY),
        ],
        out_specs=pl.BlockSpec(memory_space=pl.ANY),
        out_shape=jax.ShapeDtypeStruct((N_COLS, N_ROWS), jnp.float32),
        scratch_shapes=[
            pltpu.VMEM((8 * N_ROWS, N_ROWS), jnp.float32),
            pltpu.SemaphoreType.DMA,
        ],
        input_output_aliases={2: 0},
    )(jnp.reshape(idx, (N_ROWS,)), idx, zeros_t)
    return out_t.T


# speedup vs baseline: 1.5203x; 1.0009x over previous
import jax
import jax.numpy as jnp
import numpy as np
from jax.experimental import pallas as pl
from jax.experimental.pallas import tpu as pltpu

N_ROWS = 128
N_COLS = 100000
BLK_V = 2048
N_BLOCKS = (N_COLS + BLK_V - 1) // BLK_V


def _rotl(x, d):
    return jax.lax.shift_left(x, np.int32(d)) | jax.lax.shift_right_logical(
        x, np.int32(32 - d)
    )


def _threefry2x32_bits(flat_idx):
    ks0 = jnp.int32(0)
    ks1 = jnp.int32(42)
    ks2 = ks0 ^ ks1 ^ jnp.int32(0x1BD11BDA)
    rot_a = (13, 15, 26, 6)
    rot_b = (17, 29, 16, 24)

    def four_rounds(x0, x1, rots):
        for r in rots:
            x0 = x0 + x1
            x1 = _rotl(x1, r)
            x1 = x0 ^ x1
        return x0, x1

    x1 = flat_idx + ks1
    x0 = x1
    x1 = _rotl(x1, rot_a[0]) ^ x0
    for r in rot_a[1:]:
        x0 = x0 + x1
        x1 = _rotl(x1, r)
        x1 = x0 ^ x1
    x0 = x0 + ks1
    x1 = x1 + ks2 + jnp.int32(1)
    x0, x1 = four_rounds(x0, x1, rot_b)
    x0 = x0 + ks2
    x1 = x1 + ks0 + jnp.int32(2)
    x0, x1 = four_rounds(x0, x1, rot_a)
    x0 = x0 + ks0
    x1 = x1 + ks1 + jnp.int32(3)
    x0, x1 = four_rounds(x0, x1, rot_b)
    x0 = x0 + ks1
    x1 = x1 + ks2 + jnp.int32(4)
    x0, x1 = four_rounds(x0, x1, rot_a)
    x0 = x0 + ks2
    x1 = x1 + ks0 + jnp.int32(5)
    return x0 ^ x1


def _gumbel_noise(bits):
    fb = jax.lax.shift_right_logical(bits, np.int32(9)) | jnp.int32(0x3F800000)
    u = jax.lax.bitcast_convert_type(fb, jnp.float32) - 1.0
    return -jnp.log(-jnp.log(u + 1e-08) + 1e-08)


def _argmax_kernel(lt_ref, idx_ref, zeros_ref, vmax_ref, vidx_ref):
    j = pl.program_id(0)
    v0 = j * BLK_V
    blk = lt_ref[...].T
    r_id = jax.lax.broadcasted_iota(jnp.int32, (N_ROWS, BLK_V), 0)
    v_id = jax.lax.broadcasted_iota(jnp.int32, (N_ROWS, BLK_V), 1) + v0
    bits = _threefry2x32_bits(r_id * N_COLS + v_id)
    noise = _gumbel_noise(bits)
    z = jnp.where(v_id < N_COLS, blk + noise, -jnp.inf)
    bmax = jnp.max(z, axis=1, keepdims=True)
    bidx = jnp.min(
        jnp.where(z == bmax, v_id, jnp.int32(0x7FFFFFFF)), axis=1, keepdims=True
    )
    zeros_ref[...] = jnp.zeros((BLK_V, N_ROWS), jnp.float32)

    @pl.when(j == 0)
    def _():
        vmax_ref[...] = bmax
        vidx_ref[...] = bidx

    @pl.when(j > 0)
    def _():
        better = bmax > vmax_ref[...]
        vidx_ref[...] = jnp.where(better, bidx, vidx_ref[...])
        vmax_ref[...] = jnp.where(better, bmax, vmax_ref[...])

    @pl.when(j == pl.num_programs(0) - 1)
    def _():
        idx_ref[...] = vidx_ref[...].T


def _patch_kernel(idx_sref, idx_ref, zeros_ref, out_ref, patch_ref, sem):
    del zeros_ref
    idxv = idx_ref[...]
    s_io = jax.lax.broadcasted_iota(jnp.int32, (8, N_ROWS), 0)

    def build(r, _):
        band_start = (idx_sref[r] // 8) * 8
        patch_ref[pl.ds(r * 8, 8), :] = (s_io + band_start == idxv).astype(
            jnp.float32
        )
        return 0

    jax.lax.fori_loop(0, N_ROWS, build, 0)

    def launch(r, _):
        band_start = (idx_sref[r] // 8) * 8
        pltpu.make_async_copy(
            patch_ref.at[pl.ds(r * 8, 8), :],
            out_ref.at[pl.ds(band_start, 8), :],
            sem,
        ).start()
        return 0

    jax.lax.fori_loop(0, N_ROWS, launch, 0)

    def wait(r, _):
        pltpu.make_async_copy(
            patch_ref.at[pl.ds(r * 8, 8), :],
            out_ref.at[pl.ds((idx_sref[r] // 8) * 8, 8), :],
            sem,
        ).wait()
        return 0

    jax.lax.fori_loop(0, N_ROWS, wait, 0)


def kernel(logits):
    lt = logits.T
    idx, zeros_t = pl.pallas_call(
        _argmax_kernel,
        grid=(N_BLOCKS,),
        in_specs=[pl.BlockSpec((BLK_V, N_ROWS), lambda j: (j, 0))],
        out_specs=[
            pl.BlockSpec((1, N_ROWS), lambda j: (0, 0)),
            pl.BlockSpec((BLK_V, N_ROWS), lambda j: (j, 0)),
        ],
        out_shape=[
            jax.ShapeDtypeStruct((1, N_ROWS), jnp.int32),
            jax.ShapeDtypeStruct((N_COLS, N_ROWS), jnp.float32),
        ],
        scratch_shapes=[
            pltpu.VMEM((N_ROWS, 1), jnp.float32),
            pltpu.VMEM((N_ROWS, 1), jnp.int32),
        ],
    )(lt)
    out_t = pl.pallas_call(
        _patch_kernel,
        in_specs=[
            pl.BlockSpec(memory_space=pltpu.SMEM),
            pl.BlockSpec(memory_space=pltpu.VMEM),
            pl.BlockSpec(memory_space=pl.ANY),
        ],
        out_specs=pl.BlockSpec(memory_space=pl.ANY),
        out_shape=jax.ShapeDtypeStruct((N_COLS, N_ROWS), jnp.float32),
        scratch_shapes=[
            pltpu.VMEM((8 * N_ROWS, N_ROWS), jnp.float32),
            pltpu.SemaphoreType.DMA,
        ],
        input_output_aliases={2: 0},
    )(jnp.reshape(idx, (N_ROWS,)), idx, zeros_t)
    return out_t.T
